# baseline (speedup 1.0000x reference)
import functools

import jax
import jax.numpy as jnp
from jax import lax
from jax.experimental import pallas as pl
from jax.experimental.pallas import tpu as pltpu
from jax.experimental.pallas import tpu_sc as plsc

N = 10000
E = 160000
H = 256
NC = 2
NS = 16
R = 10240
TILE_ROWS = R // NS
EW = 128
EROWS = 1280
EPAD = EROWS * EW
EPT = EROWS // NS
NCHUNK = 8
CW = 64
CPF = H // CW
KPC = 2
ADT = jnp.bfloat16
BLK = 512
GRID = R // BLK


def _mesh():
    return plsc.VectorSubcoreMesh(
        core_axis_name="c", subcore_axis_name="s", num_cores=NC,
        num_subcores=NS)


def _deg_body(src_hbm, dst_hbm, ones_hbm, zeros_hbm, out, idx, ones_v,
              zeros_v, acc):
    cid = lax.axis_index("c")
    sid = lax.axis_index("s")
    erow0 = sid * EPT

    @pl.when(cid == 0)
    def _():
        pltpu.sync_copy(src_hbm.at[pl.ds(erow0, EPT)], idx)

    @pl.when(cid == 1)
    def _():
        pltpu.sync_copy(dst_hbm.at[pl.ds(erow0, EPT)], idx)

    pltpu.sync_copy(ones_hbm, ones_v)
    pltpu.sync_copy(zeros_hbm, zeros_v)
    r0 = sid * TILE_ROWS
    pltpu.sync_copy(zeros_v, acc.at[pl.ds(r0, TILE_ROWS)])
    plsc.subcore_barrier()

    def body(j, carry):
        pltpu.sync_copy(ones_v, acc.at[idx.at[j]], add=True)
        return carry

    lax.fori_loop(0, EPT, body, 0)
    plsc.subcore_barrier()
    pltpu.sync_copy(acc.at[pl.ds(r0, TILE_ROWS)],
                    out.at[pl.ds(cid * R + r0, TILE_ROWS)])


def _deg_kernel():
    return pl.kernel(
        _deg_body,
        out_type=jax.ShapeDtypeStruct((NC * R, 16), jnp.float32),
        mesh=_mesh(),
        scratch_types=[
            pltpu.VMEM((EPT, EW), jnp.int32),
            pltpu.VMEM((EW, 16), jnp.float32),
            pltpu.VMEM((TILE_ROWS, 16), jnp.float32),
            pltpu.VMEM_SHARED((R, 16), jnp.float32),
        ],
        compiler_params=pltpu.CompilerParams(use_tc_tiling_on_sc=False),
    )


def _scatter_inner(xk, idx_s, idx_d, gbuf0, gbuf1, acc, sem0, sem1):
    pltpu.async_copy(xk.at[idx_s.at[0]], gbuf0, sem0)

    def body(jj, carry):
        w0 = 2 * jj
        w1 = w0 + 1
        pltpu.make_async_copy(xk.at[idx_s.at[w0]], gbuf0, sem0).wait()
        pltpu.async_copy(xk.at[idx_s.at[w1]], gbuf1, sem1)
        pltpu.sync_copy(gbuf0, acc.at[idx_d.at[w0]], add=True)
        pltpu.make_async_copy(xk.at[idx_s.at[w1]], gbuf1, sem1).wait()

        @pl.when(w1 + 1 < EPT)
        def _():
            pltpu.async_copy(xk.at[idx_s.at[w1 + 1]], gbuf0, sem0)

        pltpu.sync_copy(gbuf1, acc.at[idx_d.at[w1]], add=True)
        return carry

    lax.fori_loop(0, EPT // 2, body, 0)


def _scatter_body(src_hbm, dst_hbm, zeros_hbm, x0, x1, x2, x3, out,
                  idx_s, idx_d, gbuf0, gbuf1, zeros_v, acc, sem0, sem1):
    cid = lax.axis_index("c")
    sid = lax.axis_index("s")
    erow0 = sid * EPT
    pltpu.sync_copy(src_hbm.at[pl.ds(erow0, EPT)], idx_s)
    pltpu.sync_copy(dst_hbm.at[pl.ds(erow0, EPT)], idx_d)
    pltpu.sync_copy(zeros_hbm, zeros_v)
    r0 = sid * TILE_ROWS
    for k in range(KPC):
        pltpu.sync_copy(zeros_v, acc.at[pl.ds(r0, TILE_ROWS)])
        plsc.subcore_barrier()

        @pl.when(cid == 0)
        def _(k=k):
            _scatter_inner((x0, x1)[k], idx_s, idx_d, gbuf0, gbuf1, acc,
                           sem0, sem1)

        @pl.when(cid == 1)
        def _(k=k):
            _scatter_inner((x2, x3)[k], idx_s, idx_d, gbuf0, gbuf1, acc,
                           sem0, sem1)

        plsc.subcore_barrier()
        pltpu.sync_copy(acc.at[pl.ds(r0, TILE_ROWS)],
                        out.at[pl.ds((cid * KPC + k) * R + r0, TILE_ROWS)])


def _scatter_kernel():
    return pl.kernel(
        _scatter_body,
        out_type=jax.ShapeDtypeStruct((CPF * R, CW), ADT),
        mesh=_mesh(),
        scratch_types=[
            pltpu.VMEM((EPT, EW), jnp.int32),
            pltpu.VMEM((EPT, EW), jnp.int32),
            pltpu.VMEM((EW, CW), ADT),
            pltpu.VMEM((EW, CW), ADT),
            pltpu.VMEM((TILE_ROWS, CW), ADT),
            pltpu.VMEM_SHARED((R, CW), ADT),
            pltpu.SemaphoreType.DMA,
            pltpu.SemaphoreType.DMA,
        ],
        compiler_params=pltpu.CompilerParams(use_tc_tiling_on_sc=False),
    )


def _mm_body(dego_ref, q_ref, p_ref, wq_ref, wp_ref, *x_refs):
    deg = dego_ref[:, 0]
    norm = jnp.where(deg > 0, lax.rsqrt(deg), 0.0)[:, None]
    xq = jnp.dot(q_ref[...].astype(ADT), wq_ref[...].astype(ADT),
                 preferred_element_type=jnp.float32) * norm
    xp = jnp.dot(p_ref[...].astype(ADT), wp_ref[...].astype(ADT),
                 preferred_element_type=jnp.float32) * norm
    for c in range(CPF):
        x_refs[c][...] = xq[:, c * CW:(c + 1) * CW].astype(ADT)
        x_refs[CPF + c][...] = xp[:, c * CW:(c + 1) * CW].astype(ADT)


def _mm_call(degs, q, p, Wq, Wp):
    blkm = 400
    xspec = pl.BlockSpec((blkm, CW), lambda i: (i, 0))
    return pl.pallas_call(
        _mm_body,
        grid=(N // blkm,),
        in_specs=[
            pl.BlockSpec((blkm, 16), lambda i: (i, 0)),
            pl.BlockSpec((blkm, H), lambda i: (i, 0)),
            pl.BlockSpec((blkm, H), lambda i: (i, 0)),
            pl.BlockSpec((H, H), lambda i: (0, 0)),
            pl.BlockSpec((H, H), lambda i: (0, 0)),
        ],
        out_specs=[xspec] * NCHUNK,
        out_shape=[jax.ShapeDtypeStruct((R, CW), ADT)] * NCHUNK,
    )(degs, q, p, Wq, Wp)


def _mlp_body(partsq_ref, partsp_ref, degi_ref, w1_ref, w2_ref, bq_ref,
              bp_ref, b1_ref, b2_ref, out_ref):
    i = pl.program_id(0)
    deg = degi_ref[:, 0]
    norm = jnp.where(deg > 0, lax.rsqrt(deg), 0.0)[:, None]
    aggq = jnp.concatenate(
        [partsq_ref[c].astype(jnp.float32) for c in range(CPF)], axis=1)
    aggp = jnp.concatenate(
        [partsp_ref[c].astype(jnp.float32) for c in range(CPF)], axis=1)
    zq = aggq * norm + bq_ref[...]
    zp = aggp * norm + bp_ref[...]
    z = jnp.concatenate([zq, zp], axis=1)
    h = jnp.tanh(jnp.dot(z.astype(ADT), w1_ref[...].astype(ADT),
                         preferred_element_type=jnp.float32) + b1_ref[...])
    h2 = jnp.dot(h.astype(ADT), w2_ref[...].astype(ADT),
                 preferred_element_type=jnp.float32) + b2_ref[...]
    rows = i * BLK + lax.broadcasted_iota(jnp.int32, (BLK, 1), 0)
    ssq = 0.5 * jnp.sum(jnp.where(rows < N, h2 * h2, 0.0))

    @pl.when(i == 0)
    def _():
        out_ref[...] = jnp.zeros((1, 1), jnp.float32)

    out_ref[...] += jnp.reshape(ssq, (1, 1))


def _mlp_call(partsq4, partsp4, degs, W1, W2, bq, bp, b1, b2):
    bspec = pl.BlockSpec((1, H), lambda i: (0, 0))
    pspec = pl.BlockSpec((CPF, BLK, CW), lambda i: (0, i, 0))
    return pl.pallas_call(
        _mlp_body,
        grid=(GRID,),
        in_specs=[
            pspec,
            pspec,
            pl.BlockSpec((BLK, 16), lambda i: (GRID + i, 0)),
            pl.BlockSpec((2 * H, H), lambda i: (0, 0)),
            pl.BlockSpec((H, H), lambda i: (0, 0)),
            bspec, bspec, bspec, bspec,
        ],
        out_specs=pl.BlockSpec((1, 1), lambda i: (0, 0)),
        out_shape=jax.ShapeDtypeStruct((1, 1), jnp.float32),
    )(partsq4, partsp4, degs, W1, W2, bq, bp, b1, b2)


def kernel(q, p, edge_index, Wq, bq, Wp, bp, W1, b1, W2, b2):
    src = edge_index[0].astype(jnp.int32)
    dst = edge_index[1].astype(jnp.int32)
    npad = EPAD - E
    padv = N + (jnp.arange(npad, dtype=jnp.int32) % (R - N))
    src2d = jnp.concatenate([src, padv]).reshape(EROWS, EW)
    dst2d = jnp.concatenate([dst, padv]).reshape(EROWS, EW)

    ones16 = jnp.ones((EW, 16), jnp.float32)
    zeros16 = jnp.zeros((TILE_ROWS, 16), jnp.float32)
    zerosw = jnp.zeros((TILE_ROWS, CW), ADT)

    degs = _deg_kernel()(src2d, dst2d, ones16, zeros16)

    xs = _mm_call(degs, q, p, Wq, Wp)

    sk = _scatter_kernel()
    partsq = sk(src2d, dst2d, zerosw, xs[0], xs[1], xs[2], xs[3])
    partsp = sk(src2d, dst2d, zerosw, xs[4], xs[5], xs[6], xs[7])
    partsq4 = partsq.reshape(CPF, R, CW)
    partsp4 = partsp.reshape(CPF, R, CW)

    hs = _mlp_call(partsq4, partsp4, degs, W1, W2, bq[None, :], bp[None, :],
                   b1[None, :], b2[None, :])
    return hs[0, 0]


# speedup vs baseline: 1.2234x; 1.2234x over previous
import functools

import jax
import jax.numpy as jnp
from jax import lax
from jax.experimental import pallas as pl
from jax.experimental.pallas import tpu as pltpu
from jax.experimental.pallas import tpu_sc as plsc

N = 10000
E = 160000
H = 256
NC = 2
NS = 16
R = 10240
TILE_ROWS = R // NS
EW = 128
EROWS = 1280
EPAD = EROWS * EW
EPT = EROWS // NS
NCHUNK = 4
CW = 128
CPF = H // CW
ADT = jnp.bfloat16
BLK = 512
GRID = R // BLK


def _mesh():
    return plsc.VectorSubcoreMesh(
        core_axis_name="c", subcore_axis_name="s", num_cores=NC,
        num_subcores=NS)


def _deg_body(src_hbm, dst_hbm, ones_hbm, zeros_hbm, out, idx, ones_v,
              zeros_v, acc):
    cid = lax.axis_index("c")
    sid = lax.axis_index("s")
    erow0 = sid * EPT

    @pl.when(cid == 0)
    def _():
        pltpu.sync_copy(src_hbm.at[pl.ds(erow0, EPT)], idx)

    @pl.when(cid == 1)
    def _():
        pltpu.sync_copy(dst_hbm.at[pl.ds(erow0, EPT)], idx)

    pltpu.sync_copy(ones_hbm, ones_v)
    pltpu.sync_copy(zeros_hbm, zeros_v)
    r0 = sid * TILE_ROWS
    pltpu.sync_copy(zeros_v, acc.at[pl.ds(r0, TILE_ROWS)])
    plsc.subcore_barrier()

    def body(j, carry):
        pltpu.sync_copy(ones_v, acc.at[idx.at[j]], add=True)
        return carry

    lax.fori_loop(0, EPT, body, 0)
    plsc.subcore_barrier()
    pltpu.sync_copy(acc.at[pl.ds(r0, TILE_ROWS)],
                    out.at[pl.ds(cid * R + r0, TILE_ROWS)])


def _deg_kernel():
    return pl.kernel(
        _deg_body,
        out_type=jax.ShapeDtypeStruct((NC * R, 16), jnp.float32),
        mesh=_mesh(),
        scratch_types=[
            pltpu.VMEM((EPT, EW), jnp.int32),
            pltpu.VMEM((EW, 16), jnp.float32),
            pltpu.VMEM((TILE_ROWS, 16), jnp.float32),
            pltpu.VMEM_SHARED((R, 16), jnp.float32),
        ],
        compiler_params=pltpu.CompilerParams(use_tc_tiling_on_sc=False),
    )


def _scatter_inner(xk, idx_s, idx_d, gbuf0, gbuf1, acc, sem0, sem1):
    pltpu.async_copy(xk.at[idx_s.at[0]], gbuf0, sem0)

    def body(jj, carry):
        w0 = 2 * jj
        w1 = w0 + 1
        pltpu.make_async_copy(xk.at[idx_s.at[w0]], gbuf0, sem0).wait()
        pltpu.async_copy(xk.at[idx_s.at[w1]], gbuf1, sem1)
        pltpu.sync_copy(gbuf0, acc.at[idx_d.at[w0]], add=True)
        pltpu.make_async_copy(xk.at[idx_s.at[w1]], gbuf1, sem1).wait()

        @pl.when(w1 + 1 < EPT)
        def _():
            pltpu.async_copy(xk.at[idx_s.at[w1 + 1]], gbuf0, sem0)

        pltpu.sync_copy(gbuf1, acc.at[idx_d.at[w1]], add=True)
        return carry

    lax.fori_loop(0, EPT // 2, body, 0)


def _scatter_body(src_hbm, dst_hbm, zeros_hbm, x0, x1, x2, x3, out,
                  idx_s, idx_d, gbuf0, gbuf1, zeros_v, acc, sem0, sem1):
    cid = lax.axis_index("c")
    sid = lax.axis_index("s")
    erow0 = sid * EPT
    pltpu.sync_copy(src_hbm.at[pl.ds(erow0, EPT)], idx_s)
    pltpu.sync_copy(dst_hbm.at[pl.ds(erow0, EPT)], idx_d)
    pltpu.sync_copy(zeros_hbm, zeros_v)
    r0 = sid * TILE_ROWS
    for k in range(CPF):
        pltpu.sync_copy(zeros_v, acc.at[pl.ds(r0, TILE_ROWS)])
        plsc.subcore_barrier()

        @pl.when(cid == 0)
        def _(k=k):
            _scatter_inner((x0, x1)[k], idx_s, idx_d, gbuf0, gbuf1, acc,
                           sem0, sem1)

        @pl.when(cid == 1)
        def _(k=k):
            _scatter_inner((x2, x3)[k], idx_s, idx_d, gbuf0, gbuf1, acc,
                           sem0, sem1)

        plsc.subcore_barrier()
        pltpu.sync_copy(acc.at[pl.ds(r0, TILE_ROWS)],
                        out.at[pl.ds((cid * CPF + k) * R + r0, TILE_ROWS)])


def _scatter_kernel():
    return pl.kernel(
        _scatter_body,
        out_type=jax.ShapeDtypeStruct((NCHUNK * R, CW), ADT),
        mesh=_mesh(),
        scratch_types=[
            pltpu.VMEM((EPT, EW), jnp.int32),
            pltpu.VMEM((EPT, EW), jnp.int32),
            pltpu.VMEM((EW, CW), ADT),
            pltpu.VMEM((EW, CW), ADT),
            pltpu.VMEM((TILE_ROWS, CW), ADT),
            pltpu.VMEM_SHARED((R, CW), ADT),
            pltpu.SemaphoreType.DMA,
            pltpu.SemaphoreType.DMA,
        ],
        compiler_params=pltpu.CompilerParams(use_tc_tiling_on_sc=False),
    )


def _mm1_body(q_ref, p_ref, wq_ref, wp_ref, yq_ref, yp_ref):
    yq_ref[...] = jnp.dot(q_ref[...].astype(ADT), wq_ref[...].astype(ADT),
                          preferred_element_type=jnp.float32).astype(ADT)
    yp_ref[...] = jnp.dot(p_ref[...].astype(ADT), wp_ref[...].astype(ADT),
                          preferred_element_type=jnp.float32).astype(ADT)


def _mm2_body(dego_ref, yq_ref, yp_ref, *x_refs):
    deg = dego_ref[:, 0]
    norm = jnp.where(deg > 0, lax.rsqrt(deg), 0.0)[:, None]
    xq = yq_ref[...].astype(jnp.float32) * norm
    xp = yp_ref[...].astype(jnp.float32) * norm
    for c in range(CPF):
        x_refs[c][...] = xq[:, c * CW:(c + 1) * CW].astype(ADT)
        x_refs[CPF + c][...] = xp[:, c * CW:(c + 1) * CW].astype(ADT)


_BLKM = 400


def _mm1_call(q, p, Wq, Wp):
    yspec = pl.BlockSpec((_BLKM, H), lambda i: (i, 0))
    return pl.pallas_call(
        _mm1_body,
        grid=(N // _BLKM,),
        in_specs=[
            pl.BlockSpec((_BLKM, H), lambda i: (i, 0)),
            pl.BlockSpec((_BLKM, H), lambda i: (i, 0)),
            pl.BlockSpec((H, H), lambda i: (0, 0)),
            pl.BlockSpec((H, H), lambda i: (0, 0)),
        ],
        out_specs=[yspec, yspec],
        out_shape=[jax.ShapeDtypeStruct((N, H), ADT)] * 2,
    )(q, p, Wq, Wp)


def _mm2_call(degs, yq, yp):
    xspec = pl.BlockSpec((_BLKM, CW), lambda i: (i, 0))
    return pl.pallas_call(
        _mm2_body,
        grid=(N // _BLKM,),
        in_specs=[
            pl.BlockSpec((_BLKM, 16), lambda i: (i, 0)),
            pl.BlockSpec((_BLKM, H), lambda i: (i, 0)),
            pl.BlockSpec((_BLKM, H), lambda i: (i, 0)),
        ],
        out_specs=[xspec] * NCHUNK,
        out_shape=[jax.ShapeDtypeStruct((R, CW), ADT)] * NCHUNK,
    )(degs, yq, yp)


def _mlp_body(parts_ref, degi_ref, w1_ref, w2_ref, bq_ref, bp_ref, b1_ref,
              b2_ref, out_ref):
    i = pl.program_id(0)
    deg = degi_ref[:, 0]
    norm = jnp.where(deg > 0, lax.rsqrt(deg), 0.0)[:, None]
    aggq = jnp.concatenate(
        [parts_ref[c].astype(jnp.float32) for c in range(CPF)], axis=1)
    aggp = jnp.concatenate(
        [parts_ref[CPF + c].astype(jnp.float32) for c in range(CPF)], axis=1)
    zq = aggq * norm + bq_ref[...]
    zp = aggp * norm + bp_ref[...]
    z = jnp.concatenate([zq, zp], axis=1)
    h = jnp.tanh(jnp.dot(z.astype(ADT), w1_ref[...].astype(ADT),
                         preferred_element_type=jnp.float32) + b1_ref[...])
    h2 = jnp.dot(h.astype(ADT), w2_ref[...].astype(ADT),
                 preferred_element_type=jnp.float32) + b2_ref[...]
    rows = i * BLK + lax.broadcasted_iota(jnp.int32, (BLK, 1), 0)
    ssq = 0.5 * jnp.sum(jnp.where(rows < N, h2 * h2, 0.0))

    @pl.when(i == 0)
    def _():
        out_ref[...] = jnp.zeros((1, 1), jnp.float32)

    out_ref[...] += jnp.reshape(ssq, (1, 1))


def _mlp_call(parts4, degs, W1, W2, bq, bp, b1, b2):
    bspec = pl.BlockSpec((1, H), lambda i: (0, 0))
    return pl.pallas_call(
        _mlp_body,
        grid=(GRID,),
        in_specs=[
            pl.BlockSpec((NCHUNK, BLK, CW), lambda i: (0, i, 0)),
            pl.BlockSpec((BLK, 16), lambda i: (GRID + i, 0)),
            pl.BlockSpec((2 * H, H), lambda i: (0, 0)),
            pl.BlockSpec((H, H), lambda i: (0, 0)),
            bspec, bspec, bspec, bspec,
        ],
        out_specs=pl.BlockSpec((1, 1), lambda i: (0, 0)),
        out_shape=jax.ShapeDtypeStruct((1, 1), jnp.float32),
    )(parts4, degs, W1, W2, bq, bp, b1, b2)


def kernel(q, p, edge_index, Wq, bq, Wp, bp, W1, b1, W2, b2):
    src = edge_index[0].astype(jnp.int32)
    dst = edge_index[1].astype(jnp.int32)
    npad = EPAD - E
    padv = N + (jnp.arange(npad, dtype=jnp.int32) % (R - N))
    src2d = jnp.concatenate([src, padv]).reshape(EROWS, EW)
    dst2d = jnp.concatenate([dst, padv]).reshape(EROWS, EW)

    ones16 = jnp.ones((EW, 16), jnp.float32)
    zeros16 = jnp.zeros((TILE_ROWS, 16), jnp.float32)
    zerosw = jnp.zeros((TILE_ROWS, CW), ADT)

    degs = _deg_kernel()(src2d, dst2d, ones16, zeros16)

    yq, yp = _mm1_call(q, p, Wq, Wp)
    xs = _mm2_call(degs, yq, yp)

    parts = _scatter_kernel()(src2d, dst2d, zerosw, *xs)
    parts4 = parts.reshape(NCHUNK, R, CW)

    hs = _mlp_call(parts4, degs, W1, W2, bq[None, :], bp[None, :],
                   b1[None, :], b2[None, :])
    return hs[0, 0]


# speedup vs baseline: 1.2351x; 1.0096x over previous
import jax
import jax.numpy as jnp
from jax import lax
from jax.experimental import pallas as pl
from jax.experimental.pallas import tpu as pltpu
from jax.experimental.pallas import tpu_sc as plsc

N = 10000
E = 160000
H = 256
NC = 2
NS = 16
R = 10240
TILE_ROWS = R // NS
EW = 128
EROWS = 1280
EPAD = EROWS * EW
EPT = EROWS // NS
NCHUNK = 4
CW = 128
CPF = H // CW
ADT = jnp.bfloat16
BLK = 512
GRID = R // BLK


def _mesh():
    return plsc.VectorSubcoreMesh(
        core_axis_name="c", subcore_axis_name="s", num_cores=NC,
        num_subcores=NS)


def _deg_body(src_hbm, dst_hbm, ones_hbm, zeros_hbm, out, idx, ones_v,
              zeros_v, acc):
    cid = lax.axis_index("c")
    sid = lax.axis_index("s")
    erow0 = sid * EPT

    @pl.when(cid == 0)
    def _():
        pltpu.sync_copy(src_hbm.at[pl.ds(erow0, EPT)], idx)

    @pl.when(cid == 1)
    def _():
        pltpu.sync_copy(dst_hbm.at[pl.ds(erow0, EPT)], idx)

    pltpu.sync_copy(ones_hbm, ones_v)
    pltpu.sync_copy(zeros_hbm, zeros_v)
    r0 = sid * TILE_ROWS
    pltpu.sync_copy(zeros_v, acc.at[pl.ds(r0, TILE_ROWS)])
    plsc.subcore_barrier()

    def body(j, carry):
        pltpu.sync_copy(ones_v, acc.at[idx.at[j]], add=True)
        return carry

    lax.fori_loop(0, EPT, body, 0)
    plsc.subcore_barrier()
    pltpu.sync_copy(acc.at[pl.ds(r0, TILE_ROWS)],
                    out.at[pl.ds(cid * R + r0, TILE_ROWS)])


def _deg_kernel():
    return pl.kernel(
        _deg_body,
        out_type=jax.ShapeDtypeStruct((NC * R, 16), jnp.float32),
        mesh=_mesh(),
        scratch_types=[
            pltpu.VMEM((EPT, EW), jnp.int32),
            pltpu.VMEM((EW, 16), jnp.float32),
            pltpu.VMEM((TILE_ROWS, 16), jnp.float32),
            pltpu.VMEM_SHARED((R, 16), jnp.float32),
        ],
        compiler_params=pltpu.CompilerParams(use_tc_tiling_on_sc=False),
    )


def _scatter_inner(xk, idx_s, idx_d, gbuf0, gbuf1, acc, sem0, sem1):
    pltpu.async_copy(xk.at[idx_s.at[0]], gbuf0, sem0)

    def body(jj, carry):
        w0 = 2 * jj
        w1 = w0 + 1
        pltpu.make_async_copy(xk.at[idx_s.at[w0]], gbuf0, sem0).wait()
        pltpu.async_copy(xk.at[idx_s.at[w1]], gbuf1, sem1)
        pltpu.sync_copy(gbuf0, acc.at[idx_d.at[w0]], add=True)
        pltpu.make_async_copy(xk.at[idx_s.at[w1]], gbuf1, sem1).wait()

        @pl.when(w1 + 1 < EPT)
        def _():
            pltpu.async_copy(xk.at[idx_s.at[w1 + 1]], gbuf0, sem0)

        pltpu.sync_copy(gbuf1, acc.at[idx_d.at[w1]], add=True)
        return carry

    lax.fori_loop(0, EPT // 2, body, 0)


def _scatter_body(src_hbm, dst_hbm, zeros_hbm, x0, x1, x2, x3, out,
                  idx_s, idx_d, gbuf0, gbuf1, zeros_v, acc, sem0, sem1):
    cid = lax.axis_index("c")
    sid = lax.axis_index("s")
    erow0 = sid * EPT
    pltpu.sync_copy(src_hbm.at[pl.ds(erow0, EPT)], idx_s)
    pltpu.sync_copy(dst_hbm.at[pl.ds(erow0, EPT)], idx_d)
    pltpu.sync_copy(zeros_hbm, zeros_v)
    r0 = sid * TILE_ROWS
    for k in range(CPF):
        pltpu.sync_copy(zeros_v, acc.at[pl.ds(r0, TILE_ROWS)])
        plsc.subcore_barrier()

        @pl.when(cid == 0)
        def _(k=k):
            _scatter_inner((x0, x1)[k], idx_s, idx_d, gbuf0, gbuf1, acc,
                           sem0, sem1)

        @pl.when(cid == 1)
        def _(k=k):
            _scatter_inner((x2, x3)[k], idx_s, idx_d, gbuf0, gbuf1, acc,
                           sem0, sem1)

        plsc.subcore_barrier()
        pltpu.sync_copy(acc.at[pl.ds(r0, TILE_ROWS)],
                        out.at[pl.ds((cid * CPF + k) * R + r0, TILE_ROWS)])


def _scatter_kernel():
    return pl.kernel(
        _scatter_body,
        out_type=jax.ShapeDtypeStruct((NCHUNK * R, CW), ADT),
        mesh=_mesh(),
        scratch_types=[
            pltpu.VMEM((EPT, EW), jnp.int32),
            pltpu.VMEM((EPT, EW), jnp.int32),
            pltpu.VMEM((EW, CW), ADT),
            pltpu.VMEM((EW, CW), ADT),
            pltpu.VMEM((TILE_ROWS, CW), ADT),
            pltpu.VMEM_SHARED((R, CW), ADT),
            pltpu.SemaphoreType.DMA,
            pltpu.SemaphoreType.DMA,
        ],
        compiler_params=pltpu.CompilerParams(use_tc_tiling_on_sc=False),
    )


def _mm_body(dego_ref, q_ref, p_ref, wq_ref, wp_ref, *x_refs):
    deg = dego_ref[:, 0]
    norm = jnp.where(deg > 0, lax.rsqrt(deg), 0.0)[:, None]
    xq = jnp.dot(q_ref[...].astype(ADT), wq_ref[...].astype(ADT),
                 preferred_element_type=jnp.float32) * norm
    xp = jnp.dot(p_ref[...].astype(ADT), wp_ref[...].astype(ADT),
                 preferred_element_type=jnp.float32) * norm
    for c in range(CPF):
        x_refs[c][...] = xq[:, c * CW:(c + 1) * CW].astype(ADT)
        x_refs[CPF + c][...] = xp[:, c * CW:(c + 1) * CW].astype(ADT)


def _mm_call(degs, q, p, Wq, Wp):
    blkm = 400
    xspec = pl.BlockSpec((blkm, CW), lambda i: (i, 0))
    return pl.pallas_call(
        _mm_body,
        grid=(N // blkm,),
        in_specs=[
            pl.BlockSpec((blkm, 16), lambda i: (i, 0)),
            pl.BlockSpec((blkm, H), lambda i: (i, 0)),
            pl.BlockSpec((blkm, H), lambda i: (i, 0)),
            pl.BlockSpec((H, H), lambda i: (0, 0)),
            pl.BlockSpec((H, H), lambda i: (0, 0)),
        ],
        out_specs=[xspec] * NCHUNK,
        out_shape=[jax.ShapeDtypeStruct((R, CW), ADT)] * NCHUNK,
    )(degs, q, p, Wq, Wp)


def _mlp_body(parts_ref, degi_ref, w1_ref, w2_ref, bq_ref, bp_ref, b1_ref,
              b2_ref, out_ref):
    i = pl.program_id(0)
    deg = degi_ref[:, 0]
    norm = jnp.where(deg > 0, lax.rsqrt(deg), 0.0)[:, None]
    aggq = jnp.concatenate(
        [parts_ref[c].astype(jnp.float32) for c in range(CPF)], axis=1)
    aggp = jnp.concatenate(
        [parts_ref[CPF + c].astype(jnp.float32) for c in range(CPF)], axis=1)
    zq = aggq * norm + bq_ref[...]
    zp = aggp * norm + bp_ref[...]
    z = jnp.concatenate([zq, zp], axis=1)
    h = jnp.tanh(jnp.dot(z.astype(ADT), w1_ref[...].astype(ADT),
                         preferred_element_type=jnp.float32) + b1_ref[...])
    h2 = jnp.dot(h.astype(ADT), w2_ref[...].astype(ADT),
                 preferred_element_type=jnp.float32) + b2_ref[...]
    rows = i * BLK + lax.broadcasted_iota(jnp.int32, (BLK, 1), 0)
    ssq = 0.5 * jnp.sum(jnp.where(rows < N, h2 * h2, 0.0))

    @pl.when(i == 0)
    def _():
        out_ref[...] = jnp.zeros((1, 1), jnp.float32)

    out_ref[...] += jnp.reshape(ssq, (1, 1))


def _mlp_call(parts4, degs, W1, W2, bq, bp, b1, b2):
    bspec = pl.BlockSpec((1, H), lambda i: (0, 0))
    return pl.pallas_call(
        _mlp_body,
        grid=(GRID,),
        in_specs=[
            pl.BlockSpec((NCHUNK, BLK, CW), lambda i: (0, i, 0)),
            pl.BlockSpec((BLK, 16), lambda i: (GRID + i, 0)),
            pl.BlockSpec((2 * H, H), lambda i: (0, 0)),
            pl.BlockSpec((H, H), lambda i: (0, 0)),
            bspec, bspec, bspec, bspec,
        ],
        out_specs=pl.BlockSpec((1, 1), lambda i: (0, 0)),
        out_shape=jax.ShapeDtypeStruct((1, 1), jnp.float32),
    )(parts4, degs, W1, W2, bq, bp, b1, b2)


def kernel(q, p, edge_index, Wq, bq, Wp, bp, W1, b1, W2, b2):
    src = edge_index[0].astype(jnp.int32)
    dst = edge_index[1].astype(jnp.int32)
    npad = EPAD - E
    padv = N + (jnp.arange(npad, dtype=jnp.int32) % (R - N))
    src2d = jnp.concatenate([src, padv]).reshape(EROWS, EW)
    dst2d = jnp.concatenate([dst, padv]).reshape(EROWS, EW)

    ones16 = jnp.ones((EW, 16), jnp.float32)
    zeros16 = jnp.zeros((TILE_ROWS, 16), jnp.float32)
    zerosw = jnp.zeros((TILE_ROWS, CW), ADT)

    degs = _deg_kernel()(src2d, dst2d, ones16, zeros16)

    xs = _mm_call(degs, q, p, Wq, Wp)

    parts = _scatter_kernel()(src2d, dst2d, zerosw, *xs)
    parts4 = parts.reshape(NCHUNK, R, CW)

    hs = _mlp_call(parts4, degs, W1, W2, bq[None, :], bp[None, :],
                   b1[None, :], b2[None, :])
    return hs[0, 0]


# speedup vs baseline: 1.2490x; 1.0112x over previous
import jax
import jax.numpy as jnp
from jax import lax
from jax.experimental import pallas as pl
from jax.experimental.pallas import tpu as pltpu
from jax.experimental.pallas import tpu_sc as plsc

N = 10000
E = 160000
H = 256
NC = 2
NS = 16
R = 10240
TILE_ROWS = R // NS
EW = 128
EROWS = 1280
EPAD = EROWS * EW
EPT = EROWS // NS
NCHUNK = 4
CW = 128
CPF = H // CW
ADT = jnp.bfloat16
BLK = 512
GRID = R // BLK


def _mesh():
    return plsc.VectorSubcoreMesh(
        core_axis_name="c", subcore_axis_name="s", num_cores=NC,
        num_subcores=NS)


def _deg_body(src_hbm, dst_hbm, ones_hbm, zeros_hbm, out, idx, ones_v,
              zeros_v, acc, sem):
    cid = lax.axis_index("c")
    sid = lax.axis_index("s")
    erow0 = sid * EPT

    @pl.when(cid == 0)
    def _():
        pltpu.sync_copy(src_hbm.at[pl.ds(erow0, EPT)], idx)

    @pl.when(cid == 1)
    def _():
        pltpu.sync_copy(dst_hbm.at[pl.ds(erow0, EPT)], idx)

    pltpu.sync_copy(ones_hbm, ones_v)
    pltpu.sync_copy(zeros_hbm, zeros_v)
    r0 = sid * TILE_ROWS
    pltpu.sync_copy(zeros_v, acc.at[pl.ds(r0, TILE_ROWS)])
    plsc.subcore_barrier()

    def body(j, carry):
        pltpu.async_copy(ones_v, acc.at[idx.at[j]], sem, add=True)
        return carry

    lax.fori_loop(0, EPT, body, 0)

    def drain(j, carry):
        pltpu.make_async_copy(ones_v, acc.at[idx.at[j]], sem).wait()
        return carry

    lax.fori_loop(0, EPT, drain, 0)
    plsc.subcore_barrier()
    pltpu.sync_copy(acc.at[pl.ds(r0, TILE_ROWS)],
                    out.at[pl.ds(cid * R + r0, TILE_ROWS)])


def _deg_kernel():
    return pl.kernel(
        _deg_body,
        out_type=jax.ShapeDtypeStruct((NC * R, 16), jnp.float32),
        mesh=_mesh(),
        scratch_types=[
            pltpu.VMEM((EPT, EW), jnp.int32),
            pltpu.VMEM((EW, 16), jnp.float32),
            pltpu.VMEM((TILE_ROWS, 16), jnp.float32),
            pltpu.VMEM_SHARED((R, 16), jnp.float32),
            pltpu.SemaphoreType.DMA,
        ],
        compiler_params=pltpu.CompilerParams(use_tc_tiling_on_sc=False),
    )


def _scatter_inner(xk, idx_s, idx_d, gbuf0, gbuf1, acc, sem0, sem1):
    pltpu.async_copy(xk.at[idx_s.at[0]], gbuf0, sem0)

    def body(jj, carry):
        w0 = 2 * jj
        w1 = w0 + 1
        pltpu.make_async_copy(xk.at[idx_s.at[w0]], gbuf0, sem0).wait()
        pltpu.async_copy(xk.at[idx_s.at[w1]], gbuf1, sem1)
        pltpu.sync_copy(gbuf0, acc.at[idx_d.at[w0]], add=True)
        pltpu.make_async_copy(xk.at[idx_s.at[w1]], gbuf1, sem1).wait()

        @pl.when(w1 + 1 < EPT)
        def _():
            pltpu.async_copy(xk.at[idx_s.at[w1 + 1]], gbuf0, sem0)

        pltpu.sync_copy(gbuf1, acc.at[idx_d.at[w1]], add=True)
        return carry

    lax.fori_loop(0, EPT // 2, body, 0)


def _scatter_body(src_hbm, dst_hbm, zeros_hbm, x0, x1, x2, x3, out,
                  idx_s, idx_d, gbuf0, gbuf1, zeros_v, acc, sem0, sem1):
    cid = lax.axis_index("c")
    sid = lax.axis_index("s")
    erow0 = sid * EPT
    pltpu.sync_copy(src_hbm.at[pl.ds(erow0, EPT)], idx_s)
    pltpu.sync_copy(dst_hbm.at[pl.ds(erow0, EPT)], idx_d)
    pltpu.sync_copy(zeros_hbm, zeros_v)
    r0 = sid * TILE_ROWS
    for k in range(CPF):
        pltpu.sync_copy(zeros_v, acc.at[pl.ds(r0, TILE_ROWS)])
        plsc.subcore_barrier()

        @pl.when(cid == 0)
        def _(k=k):
            _scatter_inner((x0, x1)[k], idx_s, idx_d, gbuf0, gbuf1, acc,
                           sem0, sem1)

        @pl.when(cid == 1)
        def _(k=k):
            _scatter_inner((x2, x3)[k], idx_s, idx_d, gbuf0, gbuf1, acc,
                           sem0, sem1)

        plsc.subcore_barrier()
        pltpu.sync_copy(acc.at[pl.ds(r0, TILE_ROWS)],
                        out.at[pl.ds((cid * CPF + k) * R + r0, TILE_ROWS)])


def _scatter_kernel():
    return pl.kernel(
        _scatter_body,
        out_type=jax.ShapeDtypeStruct((NCHUNK * R, CW), ADT),
        mesh=_mesh(),
        scratch_types=[
            pltpu.VMEM((EPT, EW), jnp.int32),
            pltpu.VMEM((EPT, EW), jnp.int32),
            pltpu.VMEM((EW, CW), ADT),
            pltpu.VMEM((EW, CW), ADT),
            pltpu.VMEM((TILE_ROWS, CW), ADT),
            pltpu.VMEM_SHARED((R, CW), ADT),
            pltpu.SemaphoreType.DMA,
            pltpu.SemaphoreType.DMA,
        ],
        compiler_params=pltpu.CompilerParams(use_tc_tiling_on_sc=False),
    )


def _mm_body(dego_ref, q_ref, p_ref, wq_ref, wp_ref, *x_refs):
    deg = dego_ref[:, 0]
    norm = jnp.where(deg > 0, lax.rsqrt(deg), 0.0)[:, None]
    xq = jnp.dot(q_ref[...].astype(ADT), wq_ref[...].astype(ADT),
                 preferred_element_type=jnp.float32) * norm
    xp = jnp.dot(p_ref[...].astype(ADT), wp_ref[...].astype(ADT),
                 preferred_element_type=jnp.float32) * norm
    for c in range(CPF):
        x_refs[c][...] = xq[:, c * CW:(c + 1) * CW].astype(ADT)
        x_refs[CPF + c][...] = xp[:, c * CW:(c + 1) * CW].astype(ADT)


def _mm_call(degs, q, p, Wq, Wp):
    blkm = 400
    xspec = pl.BlockSpec((blkm, CW), lambda i: (i, 0))
    return pl.pallas_call(
        _mm_body,
        grid=(N // blkm,),
        in_specs=[
            pl.BlockSpec((blkm, 16), lambda i: (i, 0)),
            pl.BlockSpec((blkm, H), lambda i: (i, 0)),
            pl.BlockSpec((blkm, H), lambda i: (i, 0)),
            pl.BlockSpec((H, H), lambda i: (0, 0)),
            pl.BlockSpec((H, H), lambda i: (0, 0)),
        ],
        out_specs=[xspec] * NCHUNK,
        out_shape=[jax.ShapeDtypeStruct((R, CW), ADT)] * NCHUNK,
    )(degs, q, p, Wq, Wp)


def _mlp_body(parts_ref, degi_ref, w1_ref, w2_ref, bq_ref, bp_ref, b1_ref,
              b2_ref, out_ref):
    i = pl.program_id(0)
    deg = degi_ref[:, 0]
    norm = jnp.where(deg > 0, lax.rsqrt(deg), 0.0)[:, None]
    aggq = jnp.concatenate(
        [parts_ref[c].astype(jnp.float32) for c in range(CPF)], axis=1)
    aggp = jnp.concatenate(
        [parts_ref[CPF + c].astype(jnp.float32) for c in range(CPF)], axis=1)
    zq = aggq * norm + bq_ref[...]
    zp = aggp * norm + bp_ref[...]
    z = jnp.concatenate([zq, zp], axis=1)
    h = jnp.tanh(jnp.dot(z.astype(ADT), w1_ref[...].astype(ADT),
                         preferred_element_type=jnp.float32) + b1_ref[...])
    h2 = jnp.dot(h.astype(ADT), w2_ref[...].astype(ADT),
                 preferred_element_type=jnp.float32) + b2_ref[...]
    rows = i * BLK + lax.broadcasted_iota(jnp.int32, (BLK, 1), 0)
    ssq = 0.5 * jnp.sum(jnp.where(rows < N, h2 * h2, 0.0))

    @pl.when(i == 0)
    def _():
        out_ref[...] = jnp.zeros((1, 1), jnp.float32)

    out_ref[...] += jnp.reshape(ssq, (1, 1))


def _mlp_call(parts4, degs, W1, W2, bq, bp, b1, b2):
    bspec = pl.BlockSpec((1, H), lambda i: (0, 0))
    return pl.pallas_call(
        _mlp_body,
        grid=(GRID,),
        in_specs=[
            pl.BlockSpec((NCHUNK, BLK, CW), lambda i: (0, i, 0)),
            pl.BlockSpec((BLK, 16), lambda i: (GRID + i, 0)),
            pl.BlockSpec((2 * H, H), lambda i: (0, 0)),
            pl.BlockSpec((H, H), lambda i: (0, 0)),
            bspec, bspec, bspec, bspec,
        ],
        out_specs=pl.BlockSpec((1, 1), lambda i: (0, 0)),
        out_shape=jax.ShapeDtypeStruct((1, 1), jnp.float32),
    )(parts4, degs, W1, W2, bq, bp, b1, b2)


def kernel(q, p, edge_index, Wq, bq, Wp, bp, W1, b1, W2, b2):
    src = edge_index[0].astype(jnp.int32)
    dst = edge_index[1].astype(jnp.int32)
    npad = EPAD - E
    padv = N + (jnp.arange(npad, dtype=jnp.int32) % (R - N))
    src2d = jnp.concatenate([src, padv]).reshape(EROWS, EW)
    dst2d = jnp.concatenate([dst, padv]).reshape(EROWS, EW)

    ones16 = jnp.ones((EW, 16), jnp.float32)
    zeros16 = jnp.zeros((TILE_ROWS, 16), jnp.float32)
    zerosw = jnp.zeros((TILE_ROWS, CW), ADT)

    degs = _deg_kernel()(src2d, dst2d, ones16, zeros16)

    xs = _mm_call(degs, q, p, Wq, Wp)

    parts = _scatter_kernel()(src2d, dst2d, zerosw, *xs)
    parts4 = parts.reshape(NCHUNK, R, CW)

    hs = _mlp_call(parts4, degs, W1, W2, bq[None, :], bp[None, :],
                   b1[None, :], b2[None, :])
    return hs[0, 0]
